# baseline (speedup 1.0000x reference)
import jax
import jax.numpy as jnp
from jax.experimental import pallas as pl

MAX_LEN_ = 200
D_MODEL_ = 64
BATCH_ = 4096
BB_ = 256


def _bcast_body(pe_ref, out_ref):
    out_ref[...] = jnp.broadcast_to(pe_ref[...], out_ref.shape)


def kernel(x, pe_weight):
    batch = x.shape[0]
    flat = pe_weight.reshape(1, MAX_LEN_ * D_MODEL_)
    out = pl.pallas_call(
        _bcast_body,
        grid=(batch // BB_,),
        in_specs=[pl.BlockSpec((1, MAX_LEN_ * D_MODEL_), lambda i: (0, 0))],
        out_specs=pl.BlockSpec((BB_, MAX_LEN_ * D_MODEL_), lambda i: (i, 0)),
        out_shape=jax.ShapeDtypeStruct((batch, MAX_LEN_ * D_MODEL_), pe_weight.dtype),
    )(flat)
    return out.reshape(batch, MAX_LEN_, D_MODEL_)


# speedup vs baseline: 1.0033x; 1.0033x over previous
import jax
import jax.numpy as jnp
from jax.experimental import pallas as pl
from jax.experimental.pallas import tpu as pltpu

MAX_LEN_ = 200
D_MODEL_ = 64
ROW_ = MAX_LEN_ * D_MODEL_
K_ = 256
NCHUNK_ = 16


def _bcast_body(pe_ref, out_ref, rep_ref, sems):
    rep_ref[...] = jnp.broadcast_to(pe_ref[...], rep_ref.shape)
    for i in range(NCHUNK_):
        pltpu.make_async_copy(rep_ref, out_ref.at[pl.ds(i * K_, K_)], sems.at[i]).start()
    for i in range(NCHUNK_):
        pltpu.make_async_copy(rep_ref, out_ref.at[pl.ds(i * K_, K_)], sems.at[i]).wait()


def kernel(x, pe_weight):
    batch = x.shape[0]
    flat = pe_weight.reshape(1, ROW_)
    out = pl.pallas_call(
        _bcast_body,
        in_specs=[pl.BlockSpec(memory_space=pltpu.MemorySpace.VMEM)],
        out_specs=pl.BlockSpec(memory_space=pltpu.MemorySpace.HBM),
        out_shape=jax.ShapeDtypeStruct((batch, ROW_), pe_weight.dtype),
        scratch_shapes=[
            pltpu.VMEM((K_, ROW_), pe_weight.dtype),
            pltpu.SemaphoreType.DMA((NCHUNK_,)),
        ],
    )(flat)
    return out.reshape(batch, MAX_LEN_, D_MODEL_)
